# baseline (speedup 1.0000x reference)
import jax
import jax.numpy as jnp
from jax.experimental import pallas as pl
from jax.experimental.pallas import tpu as pltpu

_NCH = 16
_S = 4


def _dma_body(p_hbm, o_hbm, vbuf, in_sems, out_sems):
    n_batch = o_hbm.shape[0]
    ch_rows = p_hbm.shape[0] // _NCH

    def in_copy(ch):
        slot = ch % _S
        return pltpu.make_async_copy(
            p_hbm.at[pl.ds(ch * ch_rows, ch_rows)], vbuf.at[slot],
            in_sems.at[slot])

    def out_copy(ch, b):
        slot = ch % _S
        return pltpu.make_async_copy(
            vbuf.at[slot], o_hbm.at[b, pl.ds(ch * ch_rows, ch_rows)],
            out_sems.at[slot])

    for ch in range(_S):
        in_copy(ch).start()
    for ch in range(_NCH):
        in_copy(ch).wait()
        for b in range(n_batch):
            out_copy(ch, b).start()
        k = ch - (_S - 1)
        if 0 <= k and k + _S < _NCH:
            for b in range(n_batch):
                out_copy(k, b).wait()
            in_copy(k + _S).start()
    for k in range(max(0, _NCH - _S), _NCH):
        for b in range(n_batch):
            out_copy(k, b).wait()


def kernel(inputs, P):
    b, s = inputs.shape
    d = P.shape[1]
    return pl.pallas_call(
        _dma_body,
        in_specs=[pl.BlockSpec(memory_space=pltpu.MemorySpace.HBM)],
        out_specs=pl.BlockSpec(memory_space=pltpu.MemorySpace.HBM),
        out_shape=jax.ShapeDtypeStruct((b, s, d), P.dtype),
        scratch_shapes=[
            pltpu.VMEM((_S, s // _NCH, d), P.dtype),
            pltpu.SemaphoreType.DMA((_S,)),
            pltpu.SemaphoreType.DMA((_S,)),
        ],
    )(P)


# speedup vs baseline: 1.3589x; 1.3589x over previous
import jax
import jax.numpy as jnp
from jax.experimental import pallas as pl
from jax.experimental.pallas import tpu as pltpu

_NCH = 8
_S = 8


def _dma_body(p_hbm, o_hbm, vbuf, in_sems, out_sems):
    n_batch = o_hbm.shape[0]
    ch_rows = p_hbm.shape[0] // _NCH

    def in_copy(ch):
        slot = ch % _S
        return pltpu.make_async_copy(
            p_hbm.at[pl.ds(ch * ch_rows, ch_rows)], vbuf.at[slot],
            in_sems.at[slot])

    def out_copy(ch, b):
        slot = ch % _S
        return pltpu.make_async_copy(
            vbuf.at[slot], o_hbm.at[b, pl.ds(ch * ch_rows, ch_rows)],
            out_sems.at[slot])

    for ch in range(_S):
        in_copy(ch).start()
    for ch in range(_NCH):
        in_copy(ch).wait()
        for b in range(n_batch):
            out_copy(ch, b).start()
        k = ch - (_S - 1)
        if 0 <= k and k + _S < _NCH:
            for b in range(n_batch):
                out_copy(k, b).wait()
            in_copy(k + _S).start()
    for k in range(max(0, _NCH - _S), _NCH):
        for b in range(n_batch):
            out_copy(k, b).wait()


def kernel(inputs, P):
    b, s = inputs.shape
    d = P.shape[1]
    return pl.pallas_call(
        _dma_body,
        in_specs=[pl.BlockSpec(memory_space=pltpu.MemorySpace.HBM)],
        out_specs=pl.BlockSpec(memory_space=pltpu.MemorySpace.HBM),
        out_shape=jax.ShapeDtypeStruct((b, s, d), P.dtype),
        scratch_shapes=[
            pltpu.VMEM((_S, s // _NCH, d), P.dtype),
            pltpu.SemaphoreType.DMA((_S,)),
            pltpu.SemaphoreType.DMA((_S,)),
        ],
    )(P)
